# baseline (speedup 1.0000x reference)
import functools

import jax
import jax.numpy as jnp
from jax import lax
from jax.experimental import pallas as pl
from jax.experimental.pallas import tpu as pltpu
from jax.experimental.pallas import tpu_sc as plsc

H = 4
K = 8
DIM_KEY = 64
NUM_KEYS = 128
NUM_EXPERTS = 16384
HID = 1024
T = 2048
TB = 256


_NEG = -1e30


def _routing_body(x_ref, wqT_ref, keys_ref, sc0_ref, sc1_ref,
                  id0_ref, id1_ref):
    xb = x_ref[...]
    q = jnp.dot(xb, wqT_ref[...], preferred_element_type=jnp.float32)
    sims = []
    for half in range(2):
        for h in range(H):
            qs = q[:, h * 2 * DIM_KEY + half * DIM_KEY:
                   h * 2 * DIM_KEY + (half + 1) * DIM_KEY]
            kh = keys_ref[2 * h + half]
            sims.append(lax.dot_general(qs, kh, (((1,), (1,)), ((), ())),
                                        preferred_element_type=jnp.float32))
    s = jnp.concatenate(sims, axis=0)
    colf = lax.broadcasted_iota(jnp.int32, (8 * TB, NUM_KEYS), 1).astype(jnp.float32)
    ms, ps = [], []
    for _ in range(K):
        m = jnp.max(s, axis=1, keepdims=True)
        p = jnp.min(jnp.where(s == m, colf, jnp.float32(1e9)), axis=1,
                    keepdims=True)
        ms.append(m)
        ps.append(p)
        s = jnp.where(colf == p, _NEG, s)

    R = 4 * TB
    col2 = lax.broadcasted_iota(jnp.int32, (R, NUM_KEYS), 1).astype(jnp.float32)
    g = jnp.floor(col2 * 0.125)
    r = col2 - g * 8.0
    A = jnp.zeros((R, NUM_KEYS), jnp.float32)
    B = jnp.zeros((R, NUM_KEYS), jnp.float32)
    P1 = jnp.zeros((R, NUM_KEYS), jnp.float32)
    P2 = jnp.zeros((R, NUM_KEYS), jnp.float32)
    for i in range(K):
        fi = jnp.float32(i)
        A = A + jnp.where(g == fi, ms[i][:R], 0.0)
        B = B + jnp.where(r == fi, ms[i][R:], 0.0)
        P1 = P1 + jnp.where(g == fi, ps[i][:R], 0.0)
        P2 = P2 + jnp.where(r == fi, ps[i][R:], 0.0)
    comb = jnp.where(col2 < 64.0, A + B, _NEG)
    key = col2 * 16384.0 + (P1 * 128.0 + P2)
    sc_acc = jnp.zeros((R, NUM_KEYS), jnp.float32)
    id_acc = jnp.zeros((R, NUM_KEYS), jnp.float32)
    for k in range(K):
        m = jnp.max(comb, axis=1, keepdims=True)
        fkey = jnp.min(jnp.where(comb == m, key, jnp.float32(4194304.0)),
                       axis=1, keepdims=True)
        pos = jnp.floor(fkey * (1.0 / 16384.0))
        idxf = fkey - pos * 16384.0
        comb = jnp.where(col2 == pos, _NEG, comb)
        fk = jnp.float32(k)
        sc_acc = sc_acc + jnp.where(col2 == fk, m, 0.0)
        id_acc = id_acc + jnp.where(col2 == fk, idxf, 0.0)
    sc0_ref[...] = jnp.concatenate(
        [sc_acc[0:TB, 0:K], sc_acc[TB:2 * TB, 0:K]], axis=1)
    sc1_ref[...] = jnp.concatenate(
        [sc_acc[2 * TB:3 * TB, 0:K], sc_acc[3 * TB:4 * TB, 0:K]], axis=1)
    id0_ref[...] = jnp.concatenate(
        [id_acc[0:TB, 0:K], id_acc[TB:2 * TB, 0:K]], axis=1).astype(jnp.int32)
    id1_ref[...] = jnp.concatenate(
        [id_acc[2 * TB:3 * TB, 0:K], id_acc[3 * TB:4 * TB, 0:K]],
        axis=1).astype(jnp.int32)


def _routing(xs, wqT, keys_r):
    return pl.pallas_call(
        _routing_body,
        grid=(T // TB,),
        in_specs=[
            pl.BlockSpec((TB, HID), lambda i: (i, 0)),
            pl.BlockSpec((HID, 2 * DIM_KEY * H), lambda i: (0, 0)),
            pl.BlockSpec((2 * H, NUM_KEYS, DIM_KEY), lambda i: (0, 0, 0)),
        ],
        out_specs=[
            pl.BlockSpec((TB, NE), lambda i: (i, 0)),
            pl.BlockSpec((TB, NE), lambda i: (i, 0)),
            pl.BlockSpec((TB, NE), lambda i: (i, 0)),
            pl.BlockSpec((TB, NE), lambda i: (i, 0)),
        ],
        out_shape=[
            jax.ShapeDtypeStruct((T, NE), jnp.float32),
            jax.ShapeDtypeStruct((T, NE), jnp.float32),
            jax.ShapeDtypeStruct((T, NE), jnp.int32),
            jax.ShapeDtypeStruct((T, NE), jnp.int32),
        ],
    )(xs, wqT, keys_r)


NW = 32
TPW = T // NW
NE = 16
NCH = HID // 16


_GDN = lax.GatherDimensionNumbers(
    offset_dims=(), collapsed_slice_dims=(0,), start_index_map=(0,))


def _perm(v, idx):
    return lax.gather(v, idx[:, None], _GDN, (1,),
                      mode=lax.GatherScatterMode.PROMISE_IN_BOUNDS)


def _reduce16(accs, lane):
    vecs = list(accs)
    for s in (1, 2, 4, 8):
        pidx = jnp.bitwise_xor(lane, s)
        bit = jnp.bitwise_and(lane, s) != 0
        nxt = []
        for i in range(0, len(vecs), 2):
            a = vecs[i] + _perm(vecs[i], pidx)
            b = vecs[i + 1] + _perm(vecs[i + 1], pidx)
            nxt.append(jnp.where(bit, b, a))
        vecs = nxt
    return vecs[0]


def _sc_body(x_hbm, id0_hbm, id1_hbm, sc0_hbm, sc1_hbm,
             down_hbm, up_hbm, out_hbm,
             x_v, idx_all, sc_all, down_v, up_v, out_v,
             sem_x0, sem_x1, sem_d0, sem_d1, sem_u0, sem_u1,
             sem_o0, sem_o1):
    wid = lax.axis_index("s") * 2 + lax.axis_index("c")
    lane = lax.broadcasted_iota(jnp.int32, (16,), 0)
    sem_x = (sem_x0, sem_x1)
    sem_d = (sem_d0, sem_d1)
    sem_u = (sem_u0, sem_u1)
    sem_o = (sem_o0, sem_o1)
    tok0 = wid * TPW

    pltpu.sync_copy(id0_hbm.at[pl.ds(tok0, TPW)], idx_all.at[0])
    pltpu.sync_copy(id1_hbm.at[pl.ds(tok0, TPW)], idx_all.at[1])
    pltpu.sync_copy(sc0_hbm.at[pl.ds(tok0, TPW)], sc_all.at[0])
    pltpu.sync_copy(sc1_hbm.at[pl.ds(tok0, TPW)], sc_all.at[1])
    pltpu.make_async_copy(x_hbm.at[tok0], x_v.at[0], sem_x[0]).start()
    pltpu.make_async_copy(down_hbm.at[idx_all.at[0, 0]], down_v.at[0],
                          sem_d[0]).start()
    pltpu.make_async_copy(up_hbm.at[idx_all.at[0, 0]], up_v.at[0],
                          sem_u[0]).start()

    def pair_body(p, carry):
        for sub in range(2):
            t = 2 * p + sub
            tok = tok0 + t
            @pl.when(p >= 1)
            def _():
                pltpu.make_async_copy(out_v.at[sub], out_hbm.at[tok],
                                      sem_o[sub]).wait()
            pltpu.make_async_copy(down_hbm.at[idx_all.at[1, t]],
                                  down_v.at[1], sem_d[1]).start()
            pltpu.make_async_copy(up_hbm.at[idx_all.at[1, t]],
                                  up_v.at[1], sem_u[1]).start()
            if sub == 0:
                pltpu.make_async_copy(x_hbm.at[tok + 1], x_v.at[1],
                                      sem_x[1]).start()
            else:
                @pl.when(p < NPAIR - 1)
                def _():
                    pltpu.make_async_copy(x_hbm.at[tok + 1], x_v.at[0],
                                          sem_x[0]).start()
            pltpu.make_async_copy(x_hbm.at[tok], x_v.at[sub],
                                  sem_x[sub]).wait()

            for half in range(2):
                if half == 1:
                    if sub == 0:
                        pltpu.make_async_copy(
                            down_hbm.at[idx_all.at[0, t + 1]],
                            down_v.at[0], sem_d[0]).start()
                        pltpu.make_async_copy(
                            up_hbm.at[idx_all.at[0, t + 1]],
                            up_v.at[0], sem_u[0]).start()
                    else:
                        @pl.when(p < NPAIR - 1)
                        def _():
                            pltpu.make_async_copy(
                                down_hbm.at[idx_all.at[0, t + 1]],
                                down_v.at[0], sem_d[0]).start()
                            pltpu.make_async_copy(
                                up_hbm.at[idx_all.at[0, t + 1]],
                                up_v.at[0], sem_u[0]).start()
                pltpu.make_async_copy(down_hbm.at[idx_all.at[half, t]],
                                      down_v.at[half], sem_d[half]).wait()

                def dot_chunk(c, accs):
                    off = pl.multiple_of(c * 16, 16)
                    xc = x_v[sub, pl.ds(off, 16)]
                    return tuple(accs[e] + xc * down_v[half, e, pl.ds(off, 16)]
                                 for e in range(NE))

                accs = lax.fori_loop(
                    0, NCH, dot_chunk,
                    tuple(jnp.zeros((16,), jnp.float32) for _ in range(NE)))

                hvec = _reduce16(accs, lane)
                hvec = hvec * (1.0 / (1.0 + jnp.exp(-hvec)))
                hvec = hvec * jnp.maximum(sc_all[half, t], 0.0)
                splats = tuple(
                    _perm(hvec, jnp.full((16,), e, jnp.int32))
                    for e in range(NE))
                pltpu.make_async_copy(up_hbm.at[idx_all.at[half, t]],
                                      up_v.at[half], sem_u[half]).wait()

                def up_chunk(c, carry2):
                    off = pl.multiple_of(c * 16, 16)
                    if half == 0:
                        acc = jnp.zeros((16,), jnp.float32)
                    else:
                        acc = out_v[sub, pl.ds(off, 16)]
                    for e in range(NE):
                        acc = acc + splats[e] * up_v[half, e, pl.ds(off, 16)]
                    out_v[sub, pl.ds(off, 16)] = acc
                    return carry2

                lax.fori_loop(0, NCH, up_chunk, 0)
            pltpu.make_async_copy(out_v.at[sub], out_hbm.at[tok],
                                  sem_o[sub]).start()
        return carry

    lax.fori_loop(0, NPAIR, pair_body, 0)
    for sub in range(2):
        pltpu.make_async_copy(out_v.at[sub],
                              out_hbm.at[tok0 + TPW - 2 + sub],
                              sem_o[sub]).wait()


NPAIR = TPW // 2


def _sc_combine(xs, id0, id1, sc0, sc1, expert_down, expert_up):
    mesh = plsc.VectorSubcoreMesh(core_axis_name="c", subcore_axis_name="s")
    f = functools.partial(
        pl.kernel,
        mesh=mesh,
        out_type=jax.ShapeDtypeStruct((T, HID), jnp.float32),
        scratch_types=[
            pltpu.VMEM((2, HID), jnp.float32),
            pltpu.VMEM((2, TPW, NE), jnp.int32),
            pltpu.VMEM((2, TPW, NE), jnp.float32),
            pltpu.VMEM((2, NE, HID), jnp.float32),
            pltpu.VMEM((2, NE, HID), jnp.float32),
            pltpu.VMEM((2, HID), jnp.float32),
            pltpu.SemaphoreType.DMA,
            pltpu.SemaphoreType.DMA,
            pltpu.SemaphoreType.DMA,
            pltpu.SemaphoreType.DMA,
            pltpu.SemaphoreType.DMA,
            pltpu.SemaphoreType.DMA,
            pltpu.SemaphoreType.DMA,
            pltpu.SemaphoreType.DMA,
        ],
    )(_sc_body)
    return f(xs, id0, id1, sc0, sc1, expert_down, expert_up)


def kernel(x, W_q, keys, expert_down, expert_up):
    xs = x[0]
    wqT = W_q.T
    keys_r = keys.transpose(0, 2, 1, 3).reshape(2 * H, NUM_KEYS, DIM_KEY)
    sc0, sc1, id0, id1 = _routing(xs, wqT, keys_r)
    out = _sc_combine(xs, id0, id1, sc0, sc1, expert_down, expert_up)
    return out[None]


# speedup vs baseline: 1.0120x; 1.0120x over previous
import functools

import jax
import jax.numpy as jnp
from jax import lax
from jax.experimental import pallas as pl
from jax.experimental.pallas import tpu as pltpu
from jax.experimental.pallas import tpu_sc as plsc

H = 4
K = 8
DIM_KEY = 64
NUM_KEYS = 128
NUM_EXPERTS = 16384
HID = 1024
T = 2048
TB = 256


_NEG = -1e30


def _routing_body(x_ref, wqT_ref, keys_ref, sc0_ref, sc1_ref,
                  id0_ref, id1_ref):
    xb = x_ref[...]
    q = jnp.dot(xb, wqT_ref[...], preferred_element_type=jnp.float32)
    sims = []
    for half in range(2):
        for h in range(H):
            qs = q[:, h * 2 * DIM_KEY + half * DIM_KEY:
                   h * 2 * DIM_KEY + (half + 1) * DIM_KEY]
            kh = keys_ref[2 * h + half]
            sims.append(lax.dot_general(qs, kh, (((1,), (1,)), ((), ())),
                                        preferred_element_type=jnp.float32))
    s = jnp.concatenate(sims, axis=0)
    colf = lax.broadcasted_iota(jnp.int32, (8 * TB, NUM_KEYS), 1).astype(jnp.float32)
    ms, ps = [], []
    for _ in range(K):
        m = jnp.max(s, axis=1, keepdims=True)
        p = jnp.min(jnp.where(s == m, colf, jnp.float32(1e9)), axis=1,
                    keepdims=True)
        ms.append(m)
        ps.append(p)
        s = jnp.where(colf == p, _NEG, s)

    R = 4 * TB
    col2 = lax.broadcasted_iota(jnp.int32, (R, NUM_KEYS), 1).astype(jnp.float32)
    g = jnp.floor(col2 * 0.125)
    r = col2 - g * 8.0
    A = jnp.zeros((R, NUM_KEYS), jnp.float32)
    B = jnp.zeros((R, NUM_KEYS), jnp.float32)
    P1 = jnp.zeros((R, NUM_KEYS), jnp.float32)
    P2 = jnp.zeros((R, NUM_KEYS), jnp.float32)
    for i in range(K):
        fi = jnp.float32(i)
        A = A + jnp.where(g == fi, ms[i][:R], 0.0)
        B = B + jnp.where(r == fi, ms[i][R:], 0.0)
        P1 = P1 + jnp.where(g == fi, ps[i][:R], 0.0)
        P2 = P2 + jnp.where(r == fi, ps[i][R:], 0.0)
    comb = jnp.where(col2 < 64.0, A + B, _NEG)
    key = col2 * 16384.0 + (P1 * 128.0 + P2)
    sc_acc = jnp.zeros((R, NUM_KEYS), jnp.float32)
    id_acc = jnp.zeros((R, NUM_KEYS), jnp.float32)
    for k in range(K):
        m = jnp.max(comb, axis=1, keepdims=True)
        fkey = jnp.min(jnp.where(comb == m, key, jnp.float32(4194304.0)),
                       axis=1, keepdims=True)
        pos = jnp.floor(fkey * (1.0 / 16384.0))
        idxf = fkey - pos * 16384.0
        comb = jnp.where(col2 == pos, _NEG, comb)
        fk = jnp.float32(k)
        sc_acc = sc_acc + jnp.where(col2 == fk, m, 0.0)
        id_acc = id_acc + jnp.where(col2 == fk, idxf, 0.0)
    sc0_ref[...] = jnp.concatenate(
        [sc_acc[0:TB, 0:K], sc_acc[TB:2 * TB, 0:K]], axis=1)
    sc1_ref[...] = jnp.concatenate(
        [sc_acc[2 * TB:3 * TB, 0:K], sc_acc[3 * TB:4 * TB, 0:K]], axis=1)
    id0_ref[...] = jnp.concatenate(
        [id_acc[0:TB, 0:K], id_acc[TB:2 * TB, 0:K]], axis=1).astype(jnp.int32)
    id1_ref[...] = jnp.concatenate(
        [id_acc[2 * TB:3 * TB, 0:K], id_acc[3 * TB:4 * TB, 0:K]],
        axis=1).astype(jnp.int32)


def _routing(xs, wqT, keys_r):
    t_tot = xs.shape[0]
    return pl.pallas_call(
        _routing_body,
        grid=(t_tot // TB,),
        in_specs=[
            pl.BlockSpec((TB, HID), lambda i: (i, 0)),
            pl.BlockSpec((HID, 2 * DIM_KEY * H), lambda i: (0, 0)),
            pl.BlockSpec((2 * H, NUM_KEYS, DIM_KEY), lambda i: (0, 0, 0)),
        ],
        out_specs=[
            pl.BlockSpec((TB, NE), lambda i: (i, 0)),
            pl.BlockSpec((TB, NE), lambda i: (i, 0)),
            pl.BlockSpec((TB, NE), lambda i: (i, 0)),
            pl.BlockSpec((TB, NE), lambda i: (i, 0)),
        ],
        out_shape=[
            jax.ShapeDtypeStruct((t_tot, NE), jnp.float32),
            jax.ShapeDtypeStruct((t_tot, NE), jnp.float32),
            jax.ShapeDtypeStruct((t_tot, NE), jnp.int32),
            jax.ShapeDtypeStruct((t_tot, NE), jnp.int32),
        ],
    )(xs, wqT, keys_r)


NW = 32
TPW = T // NW
NE = 16
NCH = HID // 16


_GDN = lax.GatherDimensionNumbers(
    offset_dims=(), collapsed_slice_dims=(0,), start_index_map=(0,))


def _perm(v, idx):
    return lax.gather(v, idx[:, None], _GDN, (1,),
                      mode=lax.GatherScatterMode.PROMISE_IN_BOUNDS)


def _reduce16(accs, lane):
    vecs = list(accs)
    for s in (1, 2, 4, 8):
        pidx = jnp.bitwise_xor(lane, s)
        bit = jnp.bitwise_and(lane, s) != 0
        nxt = []
        for i in range(0, len(vecs), 2):
            a = vecs[i] + _perm(vecs[i], pidx)
            b = vecs[i + 1] + _perm(vecs[i + 1], pidx)
            nxt.append(jnp.where(bit, b, a))
        vecs = nxt
    return vecs[0]


def _sc_body(seg_base, tpw, x_hbm, id0_hbm, id1_hbm, sc0_hbm, sc1_hbm,
             down_hbm, up_hbm, out_hbm,
             x_v, idx_all, sc_all, down_v, up_v, out_v,
             sem_x0, sem_x1, sem_d0, sem_d1, sem_u0, sem_u1,
             sem_o0, sem_o1):
    npair = tpw // 2
    wid = lax.axis_index("s") * 2 + lax.axis_index("c")
    lane = lax.broadcasted_iota(jnp.int32, (16,), 0)
    sem_x = (sem_x0, sem_x1)
    sem_d = (sem_d0, sem_d1)
    sem_u = (sem_u0, sem_u1)
    sem_o = (sem_o0, sem_o1)
    tok0 = seg_base + wid * tpw

    pltpu.sync_copy(id0_hbm.at[pl.ds(tok0, tpw)], idx_all.at[0])
    pltpu.sync_copy(id1_hbm.at[pl.ds(tok0, tpw)], idx_all.at[1])
    pltpu.sync_copy(sc0_hbm.at[pl.ds(tok0, tpw)], sc_all.at[0])
    pltpu.sync_copy(sc1_hbm.at[pl.ds(tok0, tpw)], sc_all.at[1])
    pltpu.make_async_copy(x_hbm.at[tok0], x_v.at[0], sem_x[0]).start()
    pltpu.make_async_copy(down_hbm.at[idx_all.at[0, 0]], down_v.at[0],
                          sem_d[0]).start()
    pltpu.make_async_copy(up_hbm.at[idx_all.at[0, 0]], up_v.at[0],
                          sem_u[0]).start()

    def pair_body(p, carry):
        for sub in range(2):
            t = 2 * p + sub
            tok = tok0 + t
            @pl.when(p >= 1)
            def _():
                pltpu.make_async_copy(out_v.at[sub], out_hbm.at[tok],
                                      sem_o[sub]).wait()
            pltpu.make_async_copy(down_hbm.at[idx_all.at[1, t]],
                                  down_v.at[1], sem_d[1]).start()
            pltpu.make_async_copy(up_hbm.at[idx_all.at[1, t]],
                                  up_v.at[1], sem_u[1]).start()
            if sub == 0:
                pltpu.make_async_copy(x_hbm.at[tok + 1], x_v.at[1],
                                      sem_x[1]).start()
            else:
                @pl.when(p < npair - 1)
                def _():
                    pltpu.make_async_copy(x_hbm.at[tok + 1], x_v.at[0],
                                          sem_x[0]).start()
            pltpu.make_async_copy(x_hbm.at[tok], x_v.at[sub],
                                  sem_x[sub]).wait()

            for half in range(2):
                if half == 1:
                    if sub == 0:
                        pltpu.make_async_copy(
                            down_hbm.at[idx_all.at[0, t + 1]],
                            down_v.at[0], sem_d[0]).start()
                        pltpu.make_async_copy(
                            up_hbm.at[idx_all.at[0, t + 1]],
                            up_v.at[0], sem_u[0]).start()
                    else:
                        @pl.when(p < npair - 1)
                        def _():
                            pltpu.make_async_copy(
                                down_hbm.at[idx_all.at[0, t + 1]],
                                down_v.at[0], sem_d[0]).start()
                            pltpu.make_async_copy(
                                up_hbm.at[idx_all.at[0, t + 1]],
                                up_v.at[0], sem_u[0]).start()
                pltpu.make_async_copy(down_hbm.at[idx_all.at[half, t]],
                                      down_v.at[half], sem_d[half]).wait()

                def dot_chunk(c, accs):
                    off = pl.multiple_of(c * 16, 16)
                    xc = x_v[sub, pl.ds(off, 16)]
                    return tuple(accs[e] + xc * down_v[half, e, pl.ds(off, 16)]
                                 for e in range(NE))

                accs = lax.fori_loop(
                    0, NCH, dot_chunk,
                    tuple(jnp.zeros((16,), jnp.float32) for _ in range(NE)))

                hvec = _reduce16(accs, lane)
                hvec = hvec * (1.0 / (1.0 + jnp.exp(-hvec)))
                hvec = hvec * jnp.maximum(sc_all[half, t], 0.0)
                splats = tuple(
                    _perm(hvec, jnp.full((16,), e, jnp.int32))
                    for e in range(NE))
                pltpu.make_async_copy(up_hbm.at[idx_all.at[half, t]],
                                      up_v.at[half], sem_u[half]).wait()

                def up_chunk(c, carry2):
                    off = pl.multiple_of(c * 16, 16)
                    if half == 0:
                        acc = jnp.zeros((16,), jnp.float32)
                    else:
                        acc = out_v[sub, pl.ds(off, 16)]
                    for e in range(NE):
                        acc = acc + splats[e] * up_v[half, e, pl.ds(off, 16)]
                    out_v[sub, pl.ds(off, 16)] = acc
                    return carry2

                lax.fori_loop(0, NCH, up_chunk, 0)
            pltpu.make_async_copy(out_v.at[sub],
                                  out_hbm.at[tok - seg_base],
                                  sem_o[sub]).start()
        return carry

    lax.fori_loop(0, npair, pair_body, 0)
    for sub in range(2):
        pltpu.make_async_copy(out_v.at[sub],
                              out_hbm.at[tok0 - seg_base + tpw - 2 + sub],
                              sem_o[sub]).wait()


NPAIR = TPW // 2


def _sc_combine(xs, id0, id1, sc0, sc1, expert_down, expert_up,
                seg_base, t_seg):
    tpw = t_seg // NW
    mesh = plsc.VectorSubcoreMesh(core_axis_name="c", subcore_axis_name="s")
    f = functools.partial(
        pl.kernel,
        mesh=mesh,
        out_type=jax.ShapeDtypeStruct((t_seg, HID), jnp.float32),
        scratch_types=[
            pltpu.VMEM((2, HID), jnp.float32),
            pltpu.VMEM((2, tpw, NE), jnp.int32),
            pltpu.VMEM((2, tpw, NE), jnp.float32),
            pltpu.VMEM((2, NE, HID), jnp.float32),
            pltpu.VMEM((2, NE, HID), jnp.float32),
            pltpu.VMEM((2, HID), jnp.float32),
            pltpu.SemaphoreType.DMA,
            pltpu.SemaphoreType.DMA,
            pltpu.SemaphoreType.DMA,
            pltpu.SemaphoreType.DMA,
            pltpu.SemaphoreType.DMA,
            pltpu.SemaphoreType.DMA,
            pltpu.SemaphoreType.DMA,
            pltpu.SemaphoreType.DMA,
        ],
    )(functools.partial(_sc_body, seg_base, tpw))
    return f(xs, id0, id1, sc0, sc1, expert_down, expert_up)


def kernel(x, W_q, keys, expert_down, expert_up):
    xs = x[0]
    wqT = W_q.T
    keys_r = keys.transpose(0, 2, 1, 3).reshape(2 * H, NUM_KEYS, DIM_KEY)
    seg = T // 2
    outs = []
    for i in range(2):
        xseg = xs[i * seg:(i + 1) * seg]
        sc0, sc1, id0, id1 = _routing(xseg, wqT, keys_r)
        outs.append(_sc_combine(xseg, id0, id1, sc0, sc1,
                                expert_down, expert_up, 0, seg))
    return jnp.concatenate(outs, axis=0)[None]
